# baseline (speedup 1.0000x reference)
import functools

import jax
import jax.numpy as jnp
from jax import lax
from jax.experimental import pallas as pl
from jax.experimental.pallas import tpu as pltpu
from jax.experimental.pallas import tpu_sc as plsc

N = 10000
E = 320000
D_IN = 128
H = 128
C = 40

NC = 2
NS = 16
L = 16
NW = NC * NS
CH = 64
NCH = 158
EPW = CH * NCH
EPAD = NW * EPW
UNROLL = 6
NPAD = 10112
NPT = NPAD // NS
PADROW = N

_mesh = plsc.VectorSubcoreMesh(core_axis_name="c", subcore_axis_name="s")


@functools.partial(
    pl.kernel,
    mesh=_mesh,
    out_type=jax.ShapeDtypeStruct((NW * N,), jnp.float32),
    compiler_params=pltpu.CompilerParams(needs_layout_passes=False),
    scratch_types=[
        pltpu.VMEM((NPAD,), jnp.float32),
        pltpu.VMEM((EPW,), jnp.int32),
        pltpu.VMEM((EPW,), jnp.float32),
    ],
)
def _deg_kernel(dst_hbm, ew_hbm, out_hbm, acc, dst_v, ew_v):
    c = lax.axis_index("c")
    s = lax.axis_index("s")
    wid = s * NC + c
    base = pl.multiple_of(wid * EPW, 8)
    pltpu.sync_copy(dst_hbm.at[pl.ds(base, EPW)], dst_v)
    pltpu.sync_copy(ew_hbm.at[pl.ds(base, EPW)], ew_v)

    zero = jnp.zeros((L,), jnp.float32)

    @pl.loop(0, NPAD // L)
    def _zero(i):
        acc[pl.ds(i * L, L)] = zero

    @pl.loop(0, EPW // L)
    def _accum(i):
        idx = dst_v[pl.ds(i * L, L)]
        val = ew_v[pl.ds(i * L, L)]
        plsc.addupdate_scatter(acc, [idx], val)

    pltpu.sync_copy(acc.at[pl.ds(0, N)],
                    out_hbm.at[pl.ds(pl.multiple_of(wid * N, 8), N)])


@functools.partial(
    pl.kernel,
    mesh=_mesh,
    out_type=jax.ShapeDtypeStruct((NC, NPAD, H), jnp.float32),
    compiler_params=pltpu.CompilerParams(needs_layout_passes=False),
    scratch_types=[
        pltpu.VMEM_SHARED((NPAD, H), jnp.float32),
        pltpu.VMEM((3, CH), jnp.int32),
        pltpu.VMEM((3, CH), jnp.int32),
        pltpu.VMEM((3, CH), jnp.int32),
        pltpu.VMEM((CH, H), jnp.float32),
        pltpu.VMEM((CH, H), jnp.float32),
        pltpu.SemaphoreType.DMA,
        pltpu.SemaphoreType.DMA,
        pltpu.SemaphoreType.DMA,
        pltpu.SemaphoreType.DMA,
        pltpu.SemaphoreType.DMA,
        pltpu.SemaphoreType.DMA,
        pltpu.SemaphoreType.DMA,
    ],
)
def _edge_agg_kernel(y_hbm, e3_hbm, out_hbm,
                     accum, eb0, eb1, eb2, rows0, rows1,
                     sg0, sg1, ss0, ss1, si0, si1, si2):
    c = lax.axis_index("c")
    s = lax.axis_index("s")
    wid = s * NC + c
    row0 = wid * NCH

    ebuf = (eb0, eb1, eb2)
    rows = (rows0, rows1)
    sg = (sg0, sg1)
    ss = (ss0, ss1)
    si = (si0, si1, si2)

    def stage_idx(ci, k):
        d = k % 3
        pltpu.async_copy(e3_hbm.at[row0 + ci], ebuf[d], si[d])

    def wait_idx(ci, k):
        d = k % 3
        pltpu.make_async_copy(e3_hbm.at[row0 + ci], ebuf[d], si[d]).wait()

    def start_gather(k):
        p, d = k % 2, k % 3
        pltpu.async_copy(y_hbm.at[ebuf[d].at[0]], rows[p], sg[p])

    def wait_gather(k):
        p, d = k % 2, k % 3
        pltpu.make_async_copy(y_hbm.at[ebuf[d].at[0]], rows[p], sg[p]).wait()

    def start_scatter(k):
        p, d = k % 2, k % 3
        pltpu.async_copy(rows[p], accum.at[ebuf[d].at[1]], ss[p], add=True)

    def wait_scatter(k):
        p, d = k % 2, k % 3
        pltpu.make_async_copy(rows[p], accum.at[ebuf[d].at[1]], ss[p]).wait()

    zero = jnp.zeros((L,), jnp.float32)

    @pl.loop(0, CH)
    def _zrow(r):
        for j in range(H // L):
            rows0[r, pl.ds(j * L, L)] = zero

    rbase = pl.multiple_of(s * NPT, 8)
    for k in range(NPT // CH):
        pltpu.sync_copy(rows0, accum.at[pl.ds(rbase + k * CH, CH)])
    if NPT % CH:
        pltpu.sync_copy(rows0.at[pl.ds(0, NPT % CH)],
                        accum.at[pl.ds(rbase + (NPT // CH) * CH, NPT % CH)])
    plsc.subcore_barrier()

    def emit(ci, k):
        p, d = k % 2, k % 3

        wait_gather(k)

        @pl.when(ci + 1 < NCH)
        def _():
            @pl.when(ci >= 1)
            def _():
                wait_scatter(k + 5)
            wait_idx(ci + 1, k + 1)
            start_gather(k + 1)

        @pl.loop(0, CH // L)
        def _scale(g):
            ew16 = plsc.bitcast(ebuf[d][2, pl.ds(g * L, L)], jnp.float32)
            for u in range(L):
                w = ew16[jnp.full((L,), u, jnp.int32)]
                e = g * L + u
                for j in range(H // L):
                    sl = pl.ds(j * L, L)
                    rows[p][e, sl] = rows[p][e, sl] * w

        start_scatter(k)

        @pl.when(ci + 2 < NCH)
        def _():
            stage_idx(ci + 2, k + 2)

    pltpu.sync_copy(e3_hbm.at[row0], eb0)
    start_gather(0)
    stage_idx(1, 1)

    @pl.loop(0, NCH // UNROLL)
    def _pipe(i):
        for k in range(UNROLL):
            emit(i * UNROLL + k, k)

    for k in range(NCH % UNROLL):
        emit((NCH // UNROLL) * UNROLL + k, k)

    wait_scatter(NCH - 2)
    wait_scatter(NCH - 1)

    plsc.subcore_barrier()
    pltpu.sync_copy(accum.at[pl.ds(rbase, NPT)],
                    out_hbm.at[c, pl.ds(rbase, NPT)])


RB = 1000
GRID = N // RB


def _disb_body(degp_ref, disb_ref):
    deg = jnp.sum(degp_ref[...], axis=0) + 1.0
    dis = lax.rsqrt(deg)
    disb_ref[...] = jnp.broadcast_to(dis[:, None], (N, H))


def _disb_call(degp):
    return pl.pallas_call(
        _disb_body,
        out_shape=jax.ShapeDtypeStruct((N, H), jnp.float32),
    )(degp)


def _pre_body(x_ref, win_ref, bin_ref, w1_ref, disb_ref, y1_ref):
    h0 = jnp.maximum(
        jnp.dot(x_ref[...], win_ref[...], preferred_element_type=jnp.float32)
        + bin_ref[...], 0.0)
    y1_ref[...] = disb_ref[...] * jnp.dot(
        h0, w1_ref[...], preferred_element_type=jnp.float32)


def _pre_call(x, W_in, b_in, W1, disb):
    return pl.pallas_call(
        _pre_body,
        grid=(GRID,),
        in_specs=[
            pl.BlockSpec((RB, D_IN), lambda i: (i, 0)),
            pl.BlockSpec((D_IN, H), lambda i: (0, 0)),
            pl.BlockSpec((1, H), lambda i: (0, 0)),
            pl.BlockSpec((H, H), lambda i: (0, 0)),
            pl.BlockSpec((RB, H), lambda i: (i, 0)),
        ],
        out_specs=pl.BlockSpec((RB, H), lambda i: (i, 0)),
        out_shape=jax.ShapeDtypeStruct((N, H), jnp.float32),
    )(x, W_in, b_in.reshape(1, H), W1, disb)


def _mid_body(agg_ref, y_ref, disb_ref, b_ref, w_ref, yn_ref):
    z = agg_ref[0] + agg_ref[1] + y_ref[...]
    h = jnp.maximum(disb_ref[...] * z + b_ref[...], 0.0)
    yn_ref[...] = disb_ref[...] * jnp.dot(
        h, w_ref[...], preferred_element_type=jnp.float32)


def _mid_call(agg, y, disb, b, Wn):
    return pl.pallas_call(
        _mid_body,
        grid=(GRID,),
        in_specs=[
            pl.BlockSpec((NC, RB, H), lambda i: (0, i, 0)),
            pl.BlockSpec((RB, H), lambda i: (i, 0)),
            pl.BlockSpec((RB, H), lambda i: (i, 0)),
            pl.BlockSpec((1, H), lambda i: (0, 0)),
            pl.BlockSpec((H, H), lambda i: (0, 0)),
        ],
        out_specs=pl.BlockSpec((RB, H), lambda i: (i, 0)),
        out_shape=jax.ShapeDtypeStruct((N, H), jnp.float32),
    )(agg, y, disb, b.reshape(1, H), Wn)


def _out_body(agg_ref, y_ref, disb_ref, b_ref, wout_ref, bout_ref, o_ref):
    z = agg_ref[0] + agg_ref[1] + y_ref[...]
    h = jnp.maximum(disb_ref[...] * z + b_ref[...], 0.0)
    o_ref[...] = jnp.dot(
        h, wout_ref[...], preferred_element_type=jnp.float32) + bout_ref[...]


def _out_call(agg, y, disb, b, W_out, b_out):
    return pl.pallas_call(
        _out_body,
        grid=(GRID,),
        in_specs=[
            pl.BlockSpec((NC, RB, H), lambda i: (0, i, 0)),
            pl.BlockSpec((RB, H), lambda i: (i, 0)),
            pl.BlockSpec((RB, H), lambda i: (i, 0)),
            pl.BlockSpec((1, H), lambda i: (0, 0)),
            pl.BlockSpec((H, C), lambda i: (0, 0)),
            pl.BlockSpec((1, C), lambda i: (0, 0)),
        ],
        out_specs=pl.BlockSpec((RB, C), lambda i: (i, 0)),
        out_shape=jax.ShapeDtypeStruct((N, C), jnp.float32),
    )(agg, y, disb, b.reshape(1, H), W_out, b_out.reshape(1, C))


def kernel(x, edge_index, edge_weight, W_in, b_in, W1, b1, W2, b2, W3, b3,
           W_out, b_out):
    npad_e = EPAD - E
    pad_iota = jnp.arange(npad_e, dtype=jnp.int32)
    src = jnp.concatenate([edge_index[0], pad_iota % N])
    dst = jnp.concatenate([edge_index[1], PADROW + pad_iota % (NPAD - N)])
    ew = jnp.concatenate([edge_weight, jnp.zeros((npad_e,), jnp.float32)])
    m = NW * NCH
    e3 = jnp.stack([src.reshape(m, CH), dst.reshape(m, CH),
                    lax.bitcast_convert_type(ew, jnp.int32).reshape(m, CH)],
                   axis=1)
    degp = _deg_kernel(dst, ew).reshape(NW, N)
    disb = _disb_call(degp)
    y1 = _pre_call(x, W_in, b_in, W1, disb)
    agg1 = _edge_agg_kernel(y1, e3)[:, :N]
    y2 = _mid_call(agg1, y1, disb, b1, W2)
    agg2 = _edge_agg_kernel(y2, e3)[:, :N]
    y3 = _mid_call(agg2, y2, disb, b2, W3)
    agg3 = _edge_agg_kernel(y3, e3)[:, :N]
    return _out_call(agg3, y3, disb, b3, W_out, b_out)


# speedup vs baseline: 1.2773x; 1.2773x over previous
import functools

import jax
import jax.numpy as jnp
from jax import lax
from jax.experimental import pallas as pl
from jax.experimental.pallas import tpu as pltpu
from jax.experimental.pallas import tpu_sc as plsc

N = 10000
E = 320000
D_IN = 128
H = 128
C = 40

NC = 2
NS = 16
L = 16
NW = NC * NS
CH = 128
NCH = 79
EPW = CH * NCH
EPAD = NW * EPW
UNROLL = 6
NPAD = 10112
NPT = NPAD // NS
PADROW = N

_mesh = plsc.VectorSubcoreMesh(core_axis_name="c", subcore_axis_name="s")


@functools.partial(
    pl.kernel,
    mesh=_mesh,
    out_type=jax.ShapeDtypeStruct((NW * N,), jnp.float32),
    compiler_params=pltpu.CompilerParams(needs_layout_passes=False),
    scratch_types=[
        pltpu.VMEM((NPAD,), jnp.float32),
        pltpu.VMEM((EPW,), jnp.int32),
        pltpu.VMEM((EPW,), jnp.float32),
    ],
)
def _deg_kernel(dst_hbm, ew_hbm, out_hbm, acc, dst_v, ew_v):
    c = lax.axis_index("c")
    s = lax.axis_index("s")
    wid = s * NC + c
    base = pl.multiple_of(wid * EPW, 8)
    pltpu.sync_copy(dst_hbm.at[pl.ds(base, EPW)], dst_v)
    pltpu.sync_copy(ew_hbm.at[pl.ds(base, EPW)], ew_v)

    zero = jnp.zeros((L,), jnp.float32)

    @pl.loop(0, NPAD // L)
    def _zero(i):
        acc[pl.ds(i * L, L)] = zero

    @pl.loop(0, EPW // L)
    def _accum(i):
        idx = dst_v[pl.ds(i * L, L)]
        val = ew_v[pl.ds(i * L, L)]
        plsc.addupdate_scatter(acc, [idx], val)

    pltpu.sync_copy(acc.at[pl.ds(0, N)],
                    out_hbm.at[pl.ds(pl.multiple_of(wid * N, 8), N)])


@functools.partial(
    pl.kernel,
    mesh=_mesh,
    out_type=jax.ShapeDtypeStruct((NC, NPAD, H), jnp.float32),
    compiler_params=pltpu.CompilerParams(needs_layout_passes=False),
    scratch_types=[
        pltpu.VMEM_SHARED((NPAD, H), jnp.float32),
        pltpu.VMEM((3, CH), jnp.int32),
        pltpu.VMEM((3, CH), jnp.int32),
        pltpu.VMEM((3, CH), jnp.int32),
        pltpu.VMEM((CH, H), jnp.float32),
        pltpu.VMEM((CH, H), jnp.float32),
        pltpu.SemaphoreType.DMA,
        pltpu.SemaphoreType.DMA,
        pltpu.SemaphoreType.DMA,
        pltpu.SemaphoreType.DMA,
        pltpu.SemaphoreType.DMA,
        pltpu.SemaphoreType.DMA,
        pltpu.SemaphoreType.DMA,
    ],
)
def _edge_agg_kernel(y_hbm, e3_hbm, out_hbm,
                     accum, eb0, eb1, eb2, rows0, rows1,
                     sg0, sg1, ss0, ss1, si0, si1, si2):
    c = lax.axis_index("c")
    s = lax.axis_index("s")
    wid = s * NC + c
    row0 = wid * NCH

    ebuf = (eb0, eb1, eb2)
    rows = (rows0, rows1)
    sg = (sg0, sg1)
    ss = (ss0, ss1)
    si = (si0, si1, si2)

    def stage_idx(ci, k):
        d = k % 3
        pltpu.async_copy(e3_hbm.at[row0 + ci], ebuf[d], si[d])

    def wait_idx(ci, k):
        d = k % 3
        pltpu.make_async_copy(e3_hbm.at[row0 + ci], ebuf[d], si[d]).wait()

    def start_gather(k):
        p, d = k % 2, k % 3
        pltpu.async_copy(y_hbm.at[ebuf[d].at[0]], rows[p], sg[p])

    def wait_gather(k):
        p, d = k % 2, k % 3
        pltpu.make_async_copy(y_hbm.at[ebuf[d].at[0]], rows[p], sg[p]).wait()

    def start_scatter(k):
        p, d = k % 2, k % 3
        pltpu.async_copy(rows[p], accum.at[ebuf[d].at[1]], ss[p], add=True)

    def wait_scatter(k):
        p, d = k % 2, k % 3
        pltpu.make_async_copy(rows[p], accum.at[ebuf[d].at[1]], ss[p]).wait()

    stage_idx(0, 0)

    zero = jnp.zeros((L,), jnp.float32)

    @pl.loop(0, CH)
    def _zrow(r):
        for j in range(H // L):
            rows1[r, pl.ds(j * L, L)] = zero

    wait_idx(0, 0)
    start_gather(0)
    stage_idx(1, 1)

    rbase = pl.multiple_of(s * NPT, 8)
    for k in range(NPT // CH):
        pltpu.sync_copy(rows1, accum.at[pl.ds(rbase + k * CH, CH)])
    if NPT % CH:
        pltpu.sync_copy(rows1.at[pl.ds(0, NPT % CH)],
                        accum.at[pl.ds(rbase + (NPT // CH) * CH, NPT % CH)])
    plsc.subcore_barrier()

    def emit(ci, k):
        p, d = k % 2, k % 3

        wait_gather(k)

        @pl.when(ci + 1 < NCH)
        def _():
            @pl.when(ci >= 1)
            def _():
                wait_scatter(k + 5)
            wait_idx(ci + 1, k + 1)
            start_gather(k + 1)

        @pl.loop(0, CH // L)
        def _scale(g):
            ew16 = plsc.bitcast(ebuf[d][2, pl.ds(g * L, L)], jnp.float32)
            for u in range(L):
                w = ew16[jnp.full((L,), u, jnp.int32)]
                e = g * L + u
                for j in range(H // L):
                    sl = pl.ds(j * L, L)
                    rows[p][e, sl] = rows[p][e, sl] * w

        start_scatter(k)

        @pl.when(ci + 2 < NCH)
        def _():
            stage_idx(ci + 2, k + 2)

    @pl.loop(0, NCH // UNROLL)
    def _pipe(i):
        for k in range(UNROLL):
            emit(i * UNROLL + k, k)

    for k in range(NCH % UNROLL):
        emit((NCH // UNROLL) * UNROLL + k, k)

    wait_scatter(NCH - 2)
    wait_scatter(NCH - 1)

    plsc.subcore_barrier()
    pltpu.sync_copy(accum.at[pl.ds(rbase, NPT)],
                    out_hbm.at[c, pl.ds(rbase, NPT)])


RB = 1000
GRID = N // RB


def _disb_body(degp_ref, disb_ref):
    deg = jnp.sum(degp_ref[...], axis=0) + 1.0
    dis = lax.rsqrt(deg)
    disb_ref[...] = jnp.broadcast_to(dis[:, None], (N, H))


def _disb_call(degp):
    return pl.pallas_call(
        _disb_body,
        out_shape=jax.ShapeDtypeStruct((N, H), jnp.float32),
    )(degp)


def _pre_body(x_ref, win_ref, bin_ref, w1_ref, disb_ref, y1_ref):
    h0 = jnp.maximum(
        jnp.dot(x_ref[...], win_ref[...], preferred_element_type=jnp.float32)
        + bin_ref[...], 0.0)
    y1_ref[...] = disb_ref[...] * jnp.dot(
        h0, w1_ref[...], preferred_element_type=jnp.float32)


def _pre_call(x, W_in, b_in, W1, disb):
    return pl.pallas_call(
        _pre_body,
        grid=(GRID,),
        in_specs=[
            pl.BlockSpec((RB, D_IN), lambda i: (i, 0)),
            pl.BlockSpec((D_IN, H), lambda i: (0, 0)),
            pl.BlockSpec((1, H), lambda i: (0, 0)),
            pl.BlockSpec((H, H), lambda i: (0, 0)),
            pl.BlockSpec((RB, H), lambda i: (i, 0)),
        ],
        out_specs=pl.BlockSpec((RB, H), lambda i: (i, 0)),
        out_shape=jax.ShapeDtypeStruct((N, H), jnp.float32),
    )(x, W_in, b_in.reshape(1, H), W1, disb)


def _mid_body(agg_ref, y_ref, disb_ref, b_ref, w_ref, yn_ref):
    z = agg_ref[0] + agg_ref[1] + y_ref[...]
    h = jnp.maximum(disb_ref[...] * z + b_ref[...], 0.0)
    yn_ref[...] = disb_ref[...] * jnp.dot(
        h, w_ref[...], preferred_element_type=jnp.float32)


def _mid_call(agg, y, disb, b, Wn):
    return pl.pallas_call(
        _mid_body,
        grid=(GRID,),
        in_specs=[
            pl.BlockSpec((NC, RB, H), lambda i: (0, i, 0)),
            pl.BlockSpec((RB, H), lambda i: (i, 0)),
            pl.BlockSpec((RB, H), lambda i: (i, 0)),
            pl.BlockSpec((1, H), lambda i: (0, 0)),
            pl.BlockSpec((H, H), lambda i: (0, 0)),
        ],
        out_specs=pl.BlockSpec((RB, H), lambda i: (i, 0)),
        out_shape=jax.ShapeDtypeStruct((N, H), jnp.float32),
    )(agg, y, disb, b.reshape(1, H), Wn)


def _out_body(agg_ref, y_ref, disb_ref, b_ref, wout_ref, bout_ref, o_ref):
    z = agg_ref[0] + agg_ref[1] + y_ref[...]
    h = jnp.maximum(disb_ref[...] * z + b_ref[...], 0.0)
    o_ref[...] = jnp.dot(
        h, wout_ref[...], preferred_element_type=jnp.float32) + bout_ref[...]


def _out_call(agg, y, disb, b, W_out, b_out):
    return pl.pallas_call(
        _out_body,
        grid=(GRID,),
        in_specs=[
            pl.BlockSpec((NC, RB, H), lambda i: (0, i, 0)),
            pl.BlockSpec((RB, H), lambda i: (i, 0)),
            pl.BlockSpec((RB, H), lambda i: (i, 0)),
            pl.BlockSpec((1, H), lambda i: (0, 0)),
            pl.BlockSpec((H, C), lambda i: (0, 0)),
            pl.BlockSpec((1, C), lambda i: (0, 0)),
        ],
        out_specs=pl.BlockSpec((RB, C), lambda i: (i, 0)),
        out_shape=jax.ShapeDtypeStruct((N, C), jnp.float32),
    )(agg, y, disb, b.reshape(1, H), W_out, b_out.reshape(1, C))


def kernel(x, edge_index, edge_weight, W_in, b_in, W1, b1, W2, b2, W3, b3,
           W_out, b_out):
    npad_e = EPAD - E
    pad_iota = jnp.arange(npad_e, dtype=jnp.int32)
    src = jnp.concatenate([edge_index[0], pad_iota % N])
    dst = jnp.concatenate([edge_index[1], PADROW + pad_iota % (NPAD - N)])
    ew = jnp.concatenate([edge_weight, jnp.zeros((npad_e,), jnp.float32)])
    m = NW * NCH
    e3 = jnp.stack([src.reshape(m, CH), dst.reshape(m, CH),
                    lax.bitcast_convert_type(ew, jnp.int32).reshape(m, CH)],
                   axis=1)
    degp = _deg_kernel(dst, ew).reshape(NW, N)
    disb = _disb_call(degp)
    y1 = _pre_call(x, W_in, b_in, W1, disb)
    agg1 = _edge_agg_kernel(y1, e3)[:, :N]
    y2 = _mid_call(agg1, y1, disb, b1, W2)
    agg2 = _edge_agg_kernel(y2, e3)[:, :N]
    y3 = _mid_call(agg2, y2, disb, b2, W3)
    agg3 = _edge_agg_kernel(y3, e3)[:, :N]
    return _out_call(agg3, y3, disb, b3, W_out, b_out)
